# baseline (speedup 1.0000x reference)
import jax
import jax.numpy as jnp
from jax.experimental import pallas as pl

_B, _N, _K, _D, _C1, _C2, _NC = 32, 512, 20, 5, 30, 40, 40
_F32 = jnp.float32


def _mm(a, b):
    return jax.lax.dot_general(a, b, (((1,), (0,)), ((), ())),
                               preferred_element_type=_F32)


def _dist_topk_wfm(F, wblk_ref, msum_ref, mdup_ref, dout):
    n = F.shape[0]
    sq = jnp.sum(F * F, axis=1, keepdims=True)
    G = jax.lax.dot_general(F, F, (((1,), (1,)), ((), ())),
                            preferred_element_type=_F32)
    adj = sq + sq.reshape(1, n) - 2.0 * G
    col = jax.lax.broadcasted_iota(jnp.int32, (n, n), 1)
    onesc = jnp.ones((n, 1), _F32)

    acc = jnp.zeros((n, dout), _F32)
    for k in range(_K):
        m = jnp.min(adj, axis=1, keepdims=True)
        onehot = (adj == m).astype(_F32)
        nsel = _mm(onehot, onesc)
        def _fix(adj=adj, m=m):
            idx = jnp.min(jnp.where(adj == m, col, n), axis=1)
            return (col == idx[:, None]).astype(_F32)
        onehot = jax.lax.cond(jnp.max(nsel) > 1.5, _fix, lambda: onehot)
        nb = _mm(onehot, F)
        acc = acc + _mm(nb, wblk_ref[k])
        adj = adj + onehot * 1e30
    s = _mm(acc * acc, msum_ref[...])
    denom = jnp.sqrt(s) + 1e-8
    return acc / _mm(denom, mdup_ref[...])


def _net_body(x_ref, sig_ref, w1_ref, w2_ref, ms1_ref, md1_ref,
              ms2_ref, md2_ref, dist_ref):
    x3 = x_ref[0]
    sigc = sig_ref[...]
    norms = jnp.sqrt(jnp.sum(x3 * x3, axis=1, keepdims=True) + 1e-8)
    feat = jnp.concatenate([x3, norms * sigc, jnp.ones_like(norms)], axis=1)
    feat = feat / (jnp.sqrt(jnp.sum(feat * feat, axis=1, keepdims=True)) + 1e-8)

    fm1 = _dist_topk_wfm(feat, w1_ref, ms1_ref, md1_ref, _D * _C1)
    fm2 = _dist_topk_wfm(fm1, w2_ref, ms2_ref, md2_ref, _D * _C2)

    m = _mm(jnp.full((1, _N), 1.0 / _N, _F32), fm2)
    diff = fm2 - m
    s = _mm(diff * diff, ms2_ref[...])
    dist_ref[0] = jnp.sqrt(s + 1e-8)


def _fc_body(a_ref, w_ref, b_ref, o_ref):
    o_ref[...] = jax.lax.dot_general(
        a_ref[...], w_ref[...], (((1,), (0,)), ((), ())),
        preferred_element_type=_F32) + b_ref[...]


def kernel(inputs, sig, W1, W2, Wfc, bfc):
    w1s = jax.nn.softmax(W1, axis=0)
    w2s = jax.nn.softmax(W2, axis=0).reshape(_K, _C1, _C2)
    eye = jnp.eye(_D, dtype=_F32)
    w1blk = jnp.einsum('kc,de->kdec', w1s, eye).reshape(_K, _D, _D * _C1)
    w2blk = jnp.einsum('kco,de->kdceo', w2s, eye).reshape(_K, _D * _C1, _D * _C2)
    ms1 = jnp.tile(jnp.eye(_C1, dtype=_F32), (_D, 1))
    ms2 = jnp.tile(jnp.eye(_C2, dtype=_F32), (_D, 1))

    cspec = lambda shape: pl.BlockSpec(shape, lambda b: tuple(0 for _ in shape))
    dist = pl.pallas_call(
        _net_body,
        grid=(_B,),
        in_specs=[
            pl.BlockSpec((1, _N, 3), lambda b: (b, 0, 0)),
            cspec((_N, 1)),
            cspec((_K, _D, _D * _C1)),
            cspec((_K, _D * _C1, _D * _C2)),
            cspec((_D * _C1, _C1)),
            cspec((_C1, _D * _C1)),
            cspec((_D * _C2, _C2)),
            cspec((_C2, _D * _C2)),
        ],
        out_specs=pl.BlockSpec((1, _N, _C2), lambda b: (b, 0, 0)),
        out_shape=jax.ShapeDtypeStruct((_B, _N, _C2), _F32),
    )(inputs, sig.reshape(_N, 1), w1blk, w2blk,
      ms1, ms1.T, ms2, ms2.T)

    flat = dist.reshape(_B, _N * _C2)
    out = pl.pallas_call(
        _fc_body,
        in_specs=[
            pl.BlockSpec((_B, _N * _C2), lambda: (0, 0)),
            pl.BlockSpec((_N * _C2, _NC), lambda: (0, 0)),
            pl.BlockSpec((1, _NC), lambda: (0, 0)),
        ],
        out_specs=pl.BlockSpec((_B, _NC), lambda: (0, 0)),
        out_shape=jax.ShapeDtypeStruct((_B, _NC), _F32),
    )(flat, Wfc, bfc.reshape(1, _NC))
    return out


# speedup vs baseline: 1.3080x; 1.3080x over previous
import functools
import jax
import jax.numpy as jnp
from jax import lax
from jax.experimental import pallas as pl
from jax.experimental.pallas import tpu as pltpu, tpu_sc as plsc

_B, _N, _K, _D, _C1, _C2, _NC = 32, 512, 20, 5, 30, 40, 40
_F32 = jnp.float32
_RP = 256
_NW = 32
_CH = 128


def _mm(a, b):
    return jax.lax.dot_general(a, b, (((1,), (0,)), ((), ())),
                               preferred_element_type=_F32)


def _adj_of(F):
    sq = jnp.sum(F * F, axis=1, keepdims=True)
    G = jax.lax.dot_general(F, F, (((1,), (1,)), ((), ())),
                            preferred_element_type=_F32)
    return sq + sq.reshape(1, F.shape[0]) - 2.0 * G


def _pick_min(adj, col):
    n = adj.shape[0]
    m = jnp.min(adj, axis=1, keepdims=True)
    idx = jnp.min(jnp.where(adj == m, col, n), axis=1)
    onehot = (col == idx[:, None]).astype(_F32)
    return onehot, idx


def _net_a_body(x_ref, sig_ref, w1_ref, ms1_ref, md1_ref, fm1_ref, idx_ref):
    n = _N
    x3 = x_ref[0]
    sigc = sig_ref[...]
    norms = jnp.sqrt(jnp.sum(x3 * x3, axis=1, keepdims=True) + 1e-8)
    feat = jnp.concatenate([x3, norms * sigc, jnp.ones_like(norms)], axis=1)
    feat = feat / (jnp.sqrt(jnp.sum(feat * feat, axis=1, keepdims=True)) + 1e-8)

    col = jax.lax.broadcasted_iota(jnp.int32, (n, n), 1)

    adj = _adj_of(feat)
    acc = jnp.zeros((n, _D * _C1), _F32)
    for k in range(_K):
        onehot, _ = _pick_min(adj, col)
        acc = acc + _mm(_mm(onehot, feat), w1_ref[k])
        adj = adj + onehot * 1e30
    s = _mm(acc * acc, ms1_ref[...])
    fm1 = acc / _mm(jnp.sqrt(s) + 1e-8, md1_ref[...])

    fm1_ref[0] = jnp.concatenate([fm1, jnp.zeros((n, _RP - _D * _C1), _F32)], axis=1)

    adj = _adj_of(fm1)
    idxs = []
    for k in range(_K):
        onehot, idxv = _pick_min(adj, col)
        idxs.append(idxv[:, None])
        adj = adj + onehot * 1e30
    b = pl.program_id(0)
    idx_ref[0] = jnp.concatenate(idxs, axis=1) + _N * b


def _gather_rows(table, idx):
    tot = idx.shape[0]
    per_w = tot // _NW
    mesh = plsc.VectorSubcoreMesh(core_axis_name="c", subcore_axis_name="s")

    @functools.partial(
        pl.kernel, mesh=mesh,
        out_type=jax.ShapeDtypeStruct((tot, _RP), _F32),
        scratch_types=[
            pltpu.VMEM((per_w,), jnp.int32),
            pltpu.VMEM((_CH, _RP), _F32),
            pltpu.SemaphoreType.DMA,
        ],
    )
    def k(table_hbm, idx_hbm, out_hbm, idx_v, rows_v, sem):
        wid = lax.axis_index("s") * 2 + lax.axis_index("c")
        base = wid * per_w
        pltpu.sync_copy(idx_hbm.at[pl.ds(base, per_w)], idx_v)

        def body(i, carry):
            off = i * _CH
            pltpu.async_copy(table_hbm.at[idx_v.at[pl.ds(off, _CH)]],
                             rows_v, sem).wait()
            pltpu.sync_copy(rows_v, out_hbm.at[pl.ds(base + off, _CH)])
            return carry

        lax.fori_loop(0, per_w // _CH, body, 0)

    return k(table, idx)


def _net_c_body(nb_ref, wcat_ref, ms2_ref, md2_ref, dist_ref):
    acc = _mm(nb_ref[0], wcat_ref[...])
    s = _mm(acc * acc, ms2_ref[...])
    fm2 = acc / _mm(jnp.sqrt(s) + 1e-8, md2_ref[...])
    m = _mm(jnp.full((1, _N), 1.0 / _N, _F32), fm2)
    diff = fm2 - m
    dist_ref[0] = jnp.sqrt(_mm(diff * diff, ms2_ref[...]) + 1e-8)


def _fc_body(a_ref, w_ref, b_ref, o_ref):
    o_ref[...] = _mm(a_ref[...], w_ref[...]) + b_ref[...]


def kernel(inputs, sig, W1, W2, Wfc, bfc):
    w1s = jax.nn.softmax(W1, axis=0)
    w2s = jax.nn.softmax(W2, axis=0).reshape(_K, _C1, _C2)
    eye = jnp.eye(_D, dtype=_F32)
    w1blk = jnp.einsum('kc,de->kdec', w1s, eye).reshape(_K, _D, _D * _C1)
    w2blk = jnp.einsum('kco,de->kdceo', w2s, eye).reshape(_K, _D * _C1, _D * _C2)
    wcat = jnp.concatenate(
        [w2blk, jnp.zeros((_K, _RP - _D * _C1, _D * _C2), _F32)], axis=1
    ).reshape(_K * _RP, _D * _C2)
    ms1 = jnp.tile(jnp.eye(_C1, dtype=_F32), (_D, 1))
    ms2 = jnp.tile(jnp.eye(_C2, dtype=_F32), (_D, 1))

    cspec = lambda shape: pl.BlockSpec(shape, lambda b: tuple(0 for _ in shape))
    fm1, idx2 = pl.pallas_call(
        _net_a_body,
        grid=(_B,),
        in_specs=[
            pl.BlockSpec((1, _N, 3), lambda b: (b, 0, 0)),
            cspec((_N, 1)),
            cspec((_K, _D, _D * _C1)),
            cspec((_D * _C1, _C1)),
            cspec((_C1, _D * _C1)),
        ],
        out_specs=[
            pl.BlockSpec((1, _N, _RP), lambda b: (b, 0, 0)),
            pl.BlockSpec((1, _N, _K), lambda b: (b, 0, 0)),
        ],
        out_shape=[
            jax.ShapeDtypeStruct((_B, _N, _RP), _F32),
            jax.ShapeDtypeStruct((_B, _N, _K), jnp.int32),
        ],
    )(inputs, sig.reshape(_N, 1), w1blk, ms1, ms1.T)

    nb = _gather_rows(fm1.reshape(_B * _N, _RP), idx2.reshape(_B * _N * _K))

    dist = pl.pallas_call(
        _net_c_body,
        grid=(_B,),
        in_specs=[
            pl.BlockSpec((1, _N, _K * _RP), lambda b: (b, 0, 0)),
            cspec((_K * _RP, _D * _C2)),
            cspec((_D * _C2, _C2)),
            cspec((_C2, _D * _C2)),
        ],
        out_specs=pl.BlockSpec((1, _N, _C2), lambda b: (b, 0, 0)),
        out_shape=jax.ShapeDtypeStruct((_B, _N, _C2), _F32),
    )(nb.reshape(_B, _N, _K * _RP), wcat, ms2, ms2.T)

    flat = dist.reshape(_B, _N * _C2)
    out = pl.pallas_call(
        _fc_body,
        in_specs=[
            pl.BlockSpec((_B, _N * _C2), lambda: (0, 0)),
            pl.BlockSpec((_N * _C2, _NC), lambda: (0, 0)),
            pl.BlockSpec((1, _NC), lambda: (0, 0)),
        ],
        out_specs=pl.BlockSpec((_B, _NC), lambda: (0, 0)),
        out_shape=jax.ShapeDtypeStruct((_B, _NC), _F32),
    )(flat, Wfc, bfc.reshape(1, _NC))
    return out


# speedup vs baseline: 1.4606x; 1.1167x over previous
import functools
import jax
import jax.numpy as jnp
from jax import lax
from jax.experimental import pallas as pl
from jax.experimental.pallas import tpu as pltpu, tpu_sc as plsc

_B, _N, _K, _D, _C1, _C2, _NC = 32, 512, 20, 5, 30, 40, 40
_CB = 8
_F32 = jnp.float32
_RP = 256
_NW = 32
_CH = 128


def _mm(a, b):
    return jax.lax.dot_general(a, b, (((1,), (0,)), ((), ())),
                               preferred_element_type=_F32)


def _adj_of(F):
    sq = jnp.sum(F * F, axis=1, keepdims=True)
    G = jax.lax.dot_general(F, F, (((1,), (1,)), ((), ())),
                            preferred_element_type=_F32)
    return sq + sq.reshape(1, F.shape[0]) - 2.0 * G


def _pick_min(adj, col):
    n = adj.shape[0]
    m = jnp.min(adj, axis=1, keepdims=True)
    idx = jnp.min(jnp.where(adj == m, col, n), axis=1)
    onehot = (col == idx[:, None]).astype(_F32)
    return onehot, idx


def _net_a_body(x_ref, sig_ref, w1_ref, ms1_ref, md1_ref, fm1_ref, idx_ref):
    n = _N
    x3 = x_ref[0]
    sigc = sig_ref[...]
    norms = jnp.sqrt(jnp.sum(x3 * x3, axis=1, keepdims=True) + 1e-8)
    feat = jnp.concatenate([x3, norms * sigc, jnp.ones_like(norms)], axis=1)
    feat = feat / (jnp.sqrt(jnp.sum(feat * feat, axis=1, keepdims=True)) + 1e-8)

    col = jax.lax.broadcasted_iota(jnp.int32, (n, n), 1)

    adj = _adj_of(feat)
    acc = jnp.zeros((n, _D * _C1), _F32)
    for k in range(_K):
        onehot, _ = _pick_min(adj, col)
        acc = acc + _mm(_mm(onehot, feat), w1_ref[k])
        adj = adj + onehot * 1e30
    s = _mm(acc * acc, ms1_ref[...])
    fm1 = acc / _mm(jnp.sqrt(s) + 1e-8, md1_ref[...])

    fm1_ref[0] = jnp.concatenate([fm1, jnp.zeros((n, _RP - _D * _C1), _F32)], axis=1)

    adj = _adj_of(fm1)
    idxs = []
    for k in range(_K):
        onehot, idxv = _pick_min(adj, col)
        idxs.append(idxv[:, None])
        adj = adj + onehot * 1e30
    b = pl.program_id(0)
    idx_ref[0] = jnp.concatenate(idxs, axis=1) + _N * b


def _gather_rows(table, idx):
    tot = idx.shape[0]
    per_w = tot // _NW
    mesh = plsc.VectorSubcoreMesh(core_axis_name="c", subcore_axis_name="s")

    @functools.partial(
        pl.kernel, mesh=mesh,
        out_type=jax.ShapeDtypeStruct((tot, _RP), _F32),
        scratch_types=[
            pltpu.VMEM((per_w,), jnp.int32),
            pltpu.VMEM((_CH, _RP), _F32),
            pltpu.SemaphoreType.DMA,
        ],
    )
    def k(table_hbm, idx_hbm, out_hbm, idx_v, rows_v, sem):
        wid = lax.axis_index("s") * 2 + lax.axis_index("c")
        base = wid * per_w
        pltpu.sync_copy(idx_hbm.at[pl.ds(base, per_w)], idx_v)

        def body(i, carry):
            off = i * _CH
            pltpu.async_copy(table_hbm.at[idx_v.at[pl.ds(off, _CH)]],
                             rows_v, sem).wait()
            pltpu.sync_copy(rows_v, out_hbm.at[pl.ds(base + off, _CH)])
            return carry

        lax.fori_loop(0, per_w // _CH, body, 0)

    return k(table, idx)


def _net_c_body(nb_ref, wcat_ref, ms2_ref, md2_ref, dist_ref):
    acc = _mm(nb_ref[0], wcat_ref[...])
    s = _mm(acc * acc, ms2_ref[...])
    fm2 = acc / _mm(jnp.sqrt(s) + 1e-8, md2_ref[...])
    m = _mm(jnp.full((1, _N), 1.0 / _N, _F32), fm2)
    diff = fm2 - m
    dist_ref[0] = jnp.sqrt(_mm(diff * diff, ms2_ref[...]) + 1e-8)


def _fc_body(a_ref, w_ref, b_ref, o_ref):
    o_ref[...] = _mm(a_ref[...], w_ref[...]) + b_ref[...]


def kernel(inputs, sig, W1, W2, Wfc, bfc):
    w1s = jax.nn.softmax(W1, axis=0)
    w2s = jax.nn.softmax(W2, axis=0).reshape(_K, _C1, _C2)
    eye = jnp.eye(_D, dtype=_F32)
    w1blk = jnp.einsum('kc,de->kdec', w1s, eye).reshape(_K, _D, _D * _C1)
    w2blk = jnp.einsum('kco,de->kdceo', w2s, eye).reshape(_K, _D * _C1, _D * _C2)
    wcat = jnp.concatenate(
        [w2blk, jnp.zeros((_K, _RP - _D * _C1, _D * _C2), _F32)], axis=1
    ).reshape(_K * _RP, _D * _C2)
    ms1 = jnp.tile(jnp.eye(_C1, dtype=_F32), (_D, 1))
    ms2 = jnp.tile(jnp.eye(_C2, dtype=_F32), (_D, 1))

    cspec = lambda shape: pl.BlockSpec(shape, lambda b: tuple(0 for _ in shape))
    stage_a = pl.pallas_call(
        _net_a_body,
        grid=(_CB,),
        in_specs=[
            pl.BlockSpec((1, _N, 3), lambda b: (b, 0, 0)),
            cspec((_N, 1)),
            cspec((_K, _D, _D * _C1)),
            cspec((_D * _C1, _C1)),
            cspec((_C1, _D * _C1)),
        ],
        out_specs=[
            pl.BlockSpec((1, _N, _RP), lambda b: (b, 0, 0)),
            pl.BlockSpec((1, _N, _K), lambda b: (b, 0, 0)),
        ],
        out_shape=[
            jax.ShapeDtypeStruct((_CB, _N, _RP), _F32),
            jax.ShapeDtypeStruct((_CB, _N, _K), jnp.int32),
        ],
    )
    stage_c = pl.pallas_call(
        _net_c_body,
        grid=(_CB,),
        in_specs=[
            pl.BlockSpec((1, _N, _K * _RP), lambda b: (b, 0, 0)),
            cspec((_K * _RP, _D * _C2)),
            cspec((_D * _C2, _C2)),
            cspec((_C2, _D * _C2)),
        ],
        out_specs=pl.BlockSpec((1, _N, _C2), lambda b: (b, 0, 0)),
        out_shape=jax.ShapeDtypeStruct((_CB, _N, _C2), _F32),
    )

    sigc = sig.reshape(_N, 1)
    dists = []
    for ci in range(_B // _CB):
        xin = jax.lax.slice_in_dim(inputs, ci * _CB, (ci + 1) * _CB, axis=0)
        fm1, idx2 = stage_a(xin, sigc, w1blk, ms1, ms1.T)
        nb = _gather_rows(fm1.reshape(_CB * _N, _RP), idx2.reshape(_CB * _N * _K))
        dists.append(stage_c(nb.reshape(_CB, _N, _K * _RP), wcat, ms2, ms2.T))
    dist = jnp.concatenate(dists, axis=0)

    flat = dist.reshape(_B, _N * _C2)
    out = pl.pallas_call(
        _fc_body,
        in_specs=[
            pl.BlockSpec((_B, _N * _C2), lambda: (0, 0)),
            pl.BlockSpec((_N * _C2, _NC), lambda: (0, 0)),
            pl.BlockSpec((1, _NC), lambda: (0, 0)),
        ],
        out_specs=pl.BlockSpec((_B, _NC), lambda: (0, 0)),
        out_shape=jax.ShapeDtypeStruct((_B, _NC), _F32),
    )(flat, Wfc, bfc.reshape(1, _NC))
    return out
